# baseline (speedup 1.0000x reference)
import functools

import jax
import jax.numpy as jnp
from jax import lax
from jax.experimental import pallas as pl
from jax.experimental.pallas import tpu as pltpu
from jax.experimental.pallas import tpu_sc as plsc


def _gather_kernel(num_rows, hidden, num_workers, nb, chunk):
    mesh = plsc.VectorSubcoreMesh(core_axis_name="c", subcore_axis_name="s")
    nc = 2

    @functools.partial(
        pl.kernel,
        mesh=mesh,
        compiler_params=pltpu.CompilerParams(use_tc_tiling_on_sc=False),
        out_type=[
            jax.ShapeDtypeStruct((num_rows, hidden), jnp.float32),
            jax.ShapeDtypeStruct((num_rows,), jnp.float32),
        ],
        scratch_types=[
            pltpu.VMEM((nb, chunk), jnp.int32),
            pltpu.VMEM((chunk, hidden), jnp.float32),
            pltpu.VMEM((chunk,), jnp.float32),
            pltpu.SemaphoreType.DMA,
            pltpu.SemaphoreType.DMA,
        ],
    )
    def k(idx_hbm, w_hbm, b_hbm, outw_hbm, outb_hbm, idx_v, wbuf, bbuf, wsem, bsem):
        wid = lax.axis_index("s") * nc + lax.axis_index("c")
        base = wid * (nb * chunk)
        pltpu.sync_copy(idx_hbm.at[wid], idx_v)

        def body(j, carry):
            cw = pltpu.async_copy(w_hbm.at[idx_v.at[j]], wbuf, wsem)
            cb = pltpu.async_copy(b_hbm.at[idx_v.at[j]], bbuf, bsem)
            cw.wait()
            cb.wait()
            pltpu.sync_copy(wbuf, outw_hbm.at[pl.ds(base + j * chunk, chunk)])
            pltpu.sync_copy(bbuf, outb_hbm.at[pl.ds(base + j * chunk, chunk)])
            return carry

        lax.fori_loop(0, nb, body, 0)

    return k


def kernel(output_indices, W, b):
    batch, shortlist = output_indices.shape
    hidden = W.shape[1]
    num_rows = batch * shortlist

    num_workers = 32
    chunk = 128
    per_w = num_rows // num_workers
    nb = per_w // chunk
    assert per_w * num_workers == num_rows and nb * chunk == per_w

    idx3 = output_indices.reshape(num_workers, nb, chunk)
    k = _gather_kernel(num_rows, hidden, num_workers, nb, chunk)
    w_act, b_act = k(idx3, W, b.reshape(-1))
    return (
        w_act.reshape(batch, shortlist, hidden),
        b_act.reshape(batch, shortlist, 1),
    )


# speedup vs baseline: 1.0425x; 1.0425x over previous
import functools

import jax
import jax.numpy as jnp
from jax import lax
from jax.experimental import pallas as pl
from jax.experimental.pallas import tpu as pltpu
from jax.experimental.pallas import tpu_sc as plsc


def _gather_kernel(num_rows, hidden, num_workers, nb, chunk, nbuf):
    mesh = plsc.VectorSubcoreMesh(core_axis_name="c", subcore_axis_name="s")
    nc = 2
    per_w = nb * chunk
    ngroups = nb // nbuf
    assert ngroups * nbuf == nb

    @functools.partial(
        pl.kernel,
        mesh=mesh,
        compiler_params=pltpu.CompilerParams(use_tc_tiling_on_sc=False),
        out_type=[
            jax.ShapeDtypeStruct((num_rows, hidden), jnp.float32),
            jax.ShapeDtypeStruct((num_rows,), jnp.float32),
        ],
        scratch_types=[
            pltpu.VMEM((nb, chunk), jnp.int32),
            pltpu.VMEM((nbuf, chunk, hidden), jnp.float32),
            pltpu.VMEM((per_w,), jnp.float32),
            pltpu.SemaphoreType.DMA((nbuf,)),
            pltpu.SemaphoreType.DMA((nbuf,)),
            pltpu.SemaphoreType.DMA,
        ],
    )
    def k(idx_hbm, w_hbm, b_hbm, outw_hbm, outb_hbm, idx_v, wbuf, bbuf, gw, ow, bsem):
        wid = lax.axis_index("s") * nc + lax.axis_index("c")
        base = wid * per_w
        pltpu.sync_copy(idx_hbm.at[wid], idx_v)

        def fire_gather(j, s):
            pltpu.async_copy(w_hbm.at[idx_v.at[j]], wbuf.at[s], gw.at[s])
            pltpu.async_copy(b_hbm.at[idx_v.at[j]], bbuf.at[pl.ds(j * chunk, chunk)], bsem)

        def wait_gather(j, s):
            pltpu.make_async_copy(w_hbm.at[idx_v.at[j]], wbuf.at[s], gw.at[s]).wait()

        def fire_wb(j, s):
            pltpu.async_copy(wbuf.at[s], outw_hbm.at[pl.ds(base + j * chunk, chunk)], ow.at[s])

        def wait_wb(j, s):
            pltpu.make_async_copy(
                wbuf.at[s], outw_hbm.at[pl.ds(base + j * chunk, chunk)], ow.at[s]
            ).wait()

        for s in range(nbuf):
            fire_gather(s, s)

        def body(g, carry):
            for s in range(nbuf):
                jp = (g - 1) * nbuf + s
                wait_gather(jp, s)
                fire_wb(jp, s)
            for s in range(nbuf):
                jp = (g - 1) * nbuf + s
                wait_wb(jp, s)
                fire_gather(g * nbuf + s, s)
            return carry

        lax.fori_loop(1, ngroups, body, 0)

        for s in range(nbuf):
            j = (ngroups - 1) * nbuf + s
            wait_gather(j, s)
            fire_wb(j, s)
        for s in range(nbuf):
            wait_wb((ngroups - 1) * nbuf + s, s)

        pltpu.make_async_copy(outb_hbm.at[pl.ds(base, per_w)], bbuf, bsem).wait()
        pltpu.sync_copy(bbuf, outb_hbm.at[pl.ds(base, per_w)])

    return k


def kernel(output_indices, W, b):
    batch, shortlist = output_indices.shape
    hidden = W.shape[1]
    num_rows = batch * shortlist

    num_workers = 32
    chunk = 128
    per_w = num_rows // num_workers
    nb = per_w // chunk
    nbuf = 5
    assert per_w * num_workers == num_rows and nb * chunk == per_w

    idx3 = output_indices.reshape(num_workers, nb, chunk)
    k = _gather_kernel(num_rows, hidden, num_workers, nb, chunk, nbuf)
    w_act, b_act = k(idx3, W, b.reshape(-1))
    return (
        w_act.reshape(batch, shortlist, hidden),
        b_act.reshape(batch, shortlist, 1),
    )
